# baseline (speedup 1.0000x reference)
import functools
import math

import jax
import jax.numpy as jnp
from jax import lax
from jax.experimental import pallas as pl
from jax.experimental.pallas import tpu as pltpu
from jax.experimental.pallas import tpu_sc as plsc


def _pe_table(L, E):
  R = L * E // 128

  def body(o_ref):
    r = lax.broadcasted_iota(jnp.int32, (R, 128), 0)
    c = lax.broadcasted_iota(jnp.int32, (R, 128), 1)
    flat = r * 128 + c
    l = flat // E
    e = flat % E
    pos = l.astype(jnp.float32) + 1.0
    expnt = (2 * (e >> 1)).astype(jnp.float32) * (math.log(10000.0) / E)
    ang = pos * jnp.exp(-expnt)
    o_ref[...] = jnp.where(e % 2 == 0, jnp.cos(ang), jnp.sin(ang))

  return pl.pallas_call(
      body, out_shape=jax.ShapeDtypeStruct((R, 128), jnp.float32))()


@functools.cache
def _make_emb(B, L, E):
  info = plsc.get_sparse_core_info()
  NC, NS = info.num_cores, info.num_subcores
  NW = NC * NS
  assert B % NW == 0 and E == 64
  seq_per_w = B // NW
  RL = L * E // 128
  chunks = [(o, min(128, L - o)) for o in range(0, L, 128)]
  mesh = plsc.VectorSubcoreMesh(core_axis_name="c", subcore_axis_name="s")

  @functools.partial(
      pl.kernel,
      out_type=jax.ShapeDtypeStruct((B, RL, 128), jnp.float32),
      mesh=mesh,
      scratch_types=[
          pltpu.VMEM((L,), jnp.int32),
          pltpu.VMEM((L, 128), jnp.float32),
          pltpu.VMEM((RL, 128), jnp.float32),
          pltpu.VMEM_SHARED((RL, 128), jnp.float32),
          pltpu.SemaphoreType.DMA,
      ],
  )
  def emb(x_hbm, w_hbm, pe_hbm, out_hbm, idx_v, rows_v, out_v, pe_sh, sem):
    sid = lax.axis_index("s")
    wid = sid * NC + lax.axis_index("c")
    seq0 = wid * seq_per_w

    @pl.when(sid == 0)
    def _fill_pe():
      pltpu.sync_copy(pe_hbm, pe_sh)

    plsc.subcore_barrier()

    def seq_body(i, carry):
      seq = seq0 + i
      pltpu.sync_copy(x_hbm.at[pl.ds(seq * L, L)], idx_v)
      pltpu.sync_copy(pe_sh, out_v)
      cps = [
          pltpu.async_copy(
              w_hbm.at[idx_v.at[pl.ds(o, n)]], rows_v.at[pl.ds(o, n)], sem)
          for (o, n) in chunks
      ]
      for cp in cps:
        cp.wait()

      def l_body(l, c):
        r = l // 2
        co = (l % 2) * E
        for jj in range(E // 16):
          sl = pl.ds(co + jj * 16, 16)
          plsc.addupdate(out_v.at[r, sl], rows_v[l, pl.ds(jj * 16, 16)])
        return c

      lax.fori_loop(0, L, l_body, 0)
      pltpu.sync_copy(out_v, out_hbm.at[seq])
      return carry

    lax.fori_loop(0, seq_per_w, seq_body, 0)

  return emb


def kernel(x_batch, W):
  B, L = x_batch.shape
  _, E = W.shape
  pe = _pe_table(L, E)
  x = x_batch.astype(jnp.int32).reshape(B * L)
  W128 = jnp.pad(W, ((0, 0), (0, 128 - E)))
  out = _make_emb(B, L, E)(x, W128, pe)
  return out.reshape(B, L, E)


# speedup vs baseline: 1.0050x; 1.0050x over previous
import functools
import math

import jax
import jax.numpy as jnp
from jax import lax
from jax.experimental import pallas as pl
from jax.experimental.pallas import tpu as pltpu
from jax.experimental.pallas import tpu_sc as plsc

_CHUNK = 256
_ROWS = _CHUNK // 2


def _pe_table(L, E, rows):
  def body(o_ref):
    r = lax.broadcasted_iota(jnp.int32, (rows, 128), 0)
    c = lax.broadcasted_iota(jnp.int32, (rows, 128), 1)
    flat = r * 128 + c
    l = (flat // E) % L
    e = flat % E
    pos = l.astype(jnp.float32) + 1.0
    expnt = (2 * (e >> 1)).astype(jnp.float32) * (math.log(10000.0) / E)
    ang = pos * jnp.exp(-expnt)
    o_ref[...] = jnp.where(e % 2 == 0, jnp.cos(ang), jnp.sin(ang))

  return pl.pallas_call(
      body, out_shape=jax.ShapeDtypeStruct((rows, 128), jnp.float32))()


@functools.cache
def _make_emb(B, L, E):
  info = plsc.get_sparse_core_info()
  NC, NS = info.num_cores, info.num_subcores
  NW = NC * NS
  T = B * L
  assert T % (NW * _CHUNK) == 0 and E == 64
  tok_per_w = T // NW
  n_chunks = tok_per_w // _CHUNK
  pe_rows = L * E // 128 + _ROWS
  mesh = plsc.VectorSubcoreMesh(core_axis_name="c", subcore_axis_name="s")

  @functools.partial(
      pl.kernel,
      out_type=jax.ShapeDtypeStruct((T * E // 128, 128), jnp.float32),
      mesh=mesh,
      scratch_types=[
          pltpu.VMEM((2, _CHUNK), jnp.int32),
          pltpu.VMEM((2, _CHUNK, 128), jnp.float32),
          pltpu.VMEM((2, _ROWS, 128), jnp.float32),
          pltpu.VMEM_SHARED((pe_rows, 128), jnp.float32),
          pltpu.SemaphoreType.DMA,
          pltpu.SemaphoreType.DMA,
      ],
  )
  def emb(x_hbm, w_hbm, pe_hbm, out_hbm, idx_v, rows_v, out_v, pe_sh,
          gsem, osem):
    sid = lax.axis_index("s")
    wid = sid * NC + lax.axis_index("c")
    tok0 = wid * tok_per_w

    @pl.when(sid == 0)
    def _fill_pe():
      pltpu.sync_copy(pe_hbm, pe_sh)

    plsc.subcore_barrier()

    def issue(k, buf):
      start = tok0 + k * _CHUNK
      pltpu.sync_copy(x_hbm.at[pl.ds(start, _CHUNK)], idx_v.at[buf])
      pltpu.async_copy(
          w_hbm.at[idx_v.at[buf].at[pl.ds(0, 128)]],
          rows_v.at[buf].at[pl.ds(0, 128)], gsem)
      pltpu.async_copy(
          w_hbm.at[idx_v.at[buf].at[pl.ds(128, 128)]],
          rows_v.at[buf].at[pl.ds(128, 128)], gsem)

    def drain_gather(buf):
      pltpu.make_async_copy(
          w_hbm.at[pl.ds(0, 128)], rows_v.at[buf].at[pl.ds(0, 128)],
          gsem).wait()
      pltpu.make_async_copy(
          w_hbm.at[pl.ds(0, 128)], rows_v.at[buf].at[pl.ds(128, 128)],
          gsem).wait()

    issue(0, 0)

    def chunk_body(k, carry):
      buf = lax.rem(k, 2)

      @pl.when(k + 1 < n_chunks)
      def _issue_next():
        issue(k + 1, 1 - buf)

      @pl.when(k >= 2)
      def _drain_out():
        pltpu.make_async_copy(
            out_v.at[buf], out_hbm.at[pl.ds(0, _ROWS)], osem).wait()

      start = tok0 + k * _CHUNK
      off = lax.rem(start, L) * E // 128
      pltpu.sync_copy(pe_sh.at[pl.ds(off, _ROWS)], out_v.at[buf])

      drain_gather(buf)

      def r_body(r, c):
        for half in range(2):
          for jj in range(E // 16):
            plsc.addupdate(
                out_v.at[buf, r, pl.ds(half * E + jj * 16, 16)],
                rows_v[buf, 2 * r + half, pl.ds(jj * 16, 16)])
        return c

      lax.fori_loop(0, _ROWS, r_body, 0)

      orow = pl.multiple_of((tok0 + k * _CHUNK) * E // 128, _ROWS)
      pltpu.async_copy(out_v.at[buf], out_hbm.at[pl.ds(orow, _ROWS)], osem)
      return carry

    lax.fori_loop(0, n_chunks, chunk_body, 0)
    for buf in range(2):
      pltpu.make_async_copy(
          out_v.at[buf], out_hbm.at[pl.ds(0, _ROWS)], osem).wait()

  return emb


def kernel(x_batch, W):
  B, L = x_batch.shape
  _, E = W.shape
  pe = _pe_table(L, E, L * E // 128 + _ROWS)
  x = x_batch.astype(jnp.int32).reshape(B * L)
  W128 = jnp.pad(W, ((0, 0), (0, 128 - E)))
  out = _make_emb(B, L, E)(x, W128, pe)
  return out.reshape(B, L, E)
